# baseline (speedup 1.0000x reference)
import jax
import jax.numpy as jnp
from jax.experimental import pallas as pl
from jax.experimental.pallas import tpu as pltpu

_NQ = 8
_K = 1024
_D = 256
_BLK = 2048


def _norms_t(a):
    t = (a * a).T
    acc = None
    for k in range(16):
        u = t[8 * k : 8 * k + 8, :] + t[128 + 8 * k : 136 + 8 * k, :]
        acc = u if acc is None else acc + u
    h = acc[0:4, :] + acc[4:8, :]
    h = h[0:2, :] + h[2:4, :]
    return h[0:1, :] + h[1:2, :]


_NS = 8
_H = _BLK // _NS


def _rvq_kernel(x_ref, cb_ref, qout_ref, idx_ref, loss_ref):
    rs = [x_ref[h * _H : (h + 1) * _H, :] for h in range(_NS)]
    qouts = [jnp.zeros_like(rs[h]) for h in range(_NS)]
    losses = []
    for q in range(_NQ):
        e = cb_ref[q]
        c = _norms_t(e)
        e1 = e.astype(jnp.bfloat16)
        r1 = e - e1.astype(jnp.float32)
        e2 = r1.astype(jnp.bfloat16)
        e3 = (r1 - e2.astype(jnp.float32)).astype(jnp.bfloat16)
        ecat = jnp.concatenate([e1, e2, e3], axis=1)
        lparts = []
        for h in range(_NS):
            r = rs[h]
            rn = _norms_t(r).T
            s = jax.lax.dot_general(
                r, e, (((1,), (1,)), ((), ())),
                preferred_element_type=jnp.float32)
            dist = rn - 2.0 * s + c
            m = jnp.min(dist, axis=-1, keepdims=True)
            iota = jax.lax.broadcasted_iota(jnp.int32, (_H, _K), 1)
            idx = jnp.min(jnp.where(dist == m, iota, _K),
                          axis=-1).astype(jnp.int32)
            oh = (iota == idx[:, None]).astype(jnp.bfloat16)
            qv3 = jax.lax.dot_general(
                oh, ecat, (((1,), (0,)), ((), ())),
                preferred_element_type=jnp.float32)
            qv = (qv3[:, :_D] + qv3[:, _D : 2 * _D]) + qv3[:, 2 * _D :]
            t = qv - r
            qst = r + t
            lparts.append(jnp.sum(t * t))
            qouts[h] = qouts[h] + qst
            rs[h] = r - qst
            idx_ref[h * _H : (h + 1) * _H, q] = idx
        lsum = lparts[0]
        for h in range(1, _NS):
            lsum = lsum + lparts[h]
        losses.append(lsum)
    for h in range(_NS):
        qout_ref[h * _H : (h + 1) * _H, :] = qouts[h]
    loss_ref[0, 0, :] = jnp.stack(losses)


def kernel(x, codebooks):
    b, n, d = x.shape
    tokens = b * n
    nb = tokens // _BLK
    flat = x.reshape(tokens, d)
    qout, idx, lossp = pl.pallas_call(
        _rvq_kernel,
        grid=(nb,),
        in_specs=[
            pl.BlockSpec((_BLK, d), lambda i: (i, 0)),
            pl.BlockSpec((_NQ, _K, d), lambda i: (0, 0, 0)),
        ],
        out_specs=[
            pl.BlockSpec((_BLK, d), lambda i: (i, 0)),
            pl.BlockSpec((_BLK, _NQ), lambda i: (i, 0)),
            pl.BlockSpec((1, 1, _NQ), lambda i: (i, 0, 0)),
        ],
        out_shape=[
            jax.ShapeDtypeStruct((tokens, d), jnp.float32),
            jax.ShapeDtypeStruct((tokens, _NQ), jnp.int32),
            jax.ShapeDtypeStruct((nb, 1, _NQ), jnp.float32),
        ],
    )(flat, codebooks)
    quantized_out = qout.reshape(b, n, d)
    all_indices = idx.reshape(b, n, _NQ)
    all_losses = jnp.sum(lossp, axis=0)[0] / float(tokens * d)
    return quantized_out, all_indices, all_losses


# speedup vs baseline: 1.0756x; 1.0756x over previous
import jax
import jax.numpy as jnp
from jax.experimental import pallas as pl
from jax.experimental.pallas import tpu as pltpu

_NQ = 8
_K = 1024
_D = 256
_BLK = 2048


def _norms_t(a):
    t = (a * a).T
    acc = None
    for k in range(16):
        u = t[8 * k : 8 * k + 8, :] + t[128 + 8 * k : 136 + 8 * k, :]
        acc = u if acc is None else acc + u
    h = acc[0:4, :] + acc[4:8, :]
    h = h[0:2, :] + h[2:4, :]
    return h[0:1, :] + h[1:2, :]


_NS = 4
_H = _BLK // _NS


def _rvq_kernel(x_ref, cb_ref, qout_ref, idx_ref, loss_ref):
    rs = [x_ref[h * _H : (h + 1) * _H, :] for h in range(_NS)]
    qouts = [jnp.zeros_like(rs[h]) for h in range(_NS)]
    losses = []
    for q in range(_NQ):
        e = cb_ref[q]
        c = _norms_t(e)
        e1 = e.astype(jnp.bfloat16)
        r1 = e - e1.astype(jnp.float32)
        e2 = r1.astype(jnp.bfloat16)
        e3 = (r1 - e2.astype(jnp.float32)).astype(jnp.bfloat16)
        ecat = jnp.concatenate([e1, e2, e3], axis=1)
        lparts = []
        for h in range(_NS):
            r = rs[h]
            rn = _norms_t(r).T
            s = jax.lax.dot_general(
                r, e, (((1,), (1,)), ((), ())),
                preferred_element_type=jnp.float32)
            dist = rn - 2.0 * s + c
            m = jnp.min(dist, axis=-1, keepdims=True)
            iota = jax.lax.broadcasted_iota(jnp.int32, (_H, _K), 1)
            idx = jnp.min(jnp.where(dist == m, iota, _K),
                          axis=-1).astype(jnp.int32)
            oh = (iota == idx[:, None]).astype(jnp.bfloat16)
            qv3 = jax.lax.dot_general(
                oh, ecat, (((1,), (0,)), ((), ())),
                preferred_element_type=jnp.float32)
            qv = (qv3[:, :_D] + qv3[:, _D : 2 * _D]) + qv3[:, 2 * _D :]
            t = qv - r
            qst = r + t
            lparts.append(jnp.sum(t * t))
            qouts[h] = qouts[h] + qst
            rs[h] = r - qst
            idx_ref[h * _H : (h + 1) * _H, q] = idx
        lsum = lparts[0]
        for h in range(1, _NS):
            lsum = lsum + lparts[h]
        losses.append(lsum)
    for h in range(_NS):
        qout_ref[h * _H : (h + 1) * _H, :] = qouts[h]
    loss_ref[0, 0, :] = jnp.stack(losses)


def kernel(x, codebooks):
    b, n, d = x.shape
    tokens = b * n
    nb = tokens // _BLK
    flat = x.reshape(tokens, d)
    qout, idx, lossp = pl.pallas_call(
        _rvq_kernel,
        grid=(nb,),
        in_specs=[
            pl.BlockSpec((_BLK, d), lambda i: (i, 0)),
            pl.BlockSpec((_NQ, _K, d), lambda i: (0, 0, 0)),
        ],
        out_specs=[
            pl.BlockSpec((_BLK, d), lambda i: (i, 0)),
            pl.BlockSpec((_BLK, _NQ), lambda i: (i, 0)),
            pl.BlockSpec((1, 1, _NQ), lambda i: (i, 0, 0)),
        ],
        out_shape=[
            jax.ShapeDtypeStruct((tokens, d), jnp.float32),
            jax.ShapeDtypeStruct((tokens, _NQ), jnp.int32),
            jax.ShapeDtypeStruct((nb, 1, _NQ), jnp.float32),
        ],
    )(flat, codebooks)
    quantized_out = qout.reshape(b, n, d)
    all_indices = idx.reshape(b, n, _NQ)
    all_losses = jnp.sum(lossp, axis=0)[0] / float(tokens * d)
    return quantized_out, all_indices, all_losses
